# Initial kernel scaffold; baseline (speedup 1.0000x reference)
import jax
import jax.numpy as jnp
from jax.experimental import pallas as pl


def kernel(gate_type, edge_index, edge_w, emb, W1, W2, b2, Wf1, bf1, Wf2, bf2):
    raise NotImplementedError("write your pallas kernel here")


# speedup vs baseline: 1.7150x; 1.7150x over previous
import functools

import jax
import jax.numpy as jnp
from jax import lax
from jax.experimental import pallas as pl
from jax.experimental.pallas import tpu as pltpu
from jax.experimental.pallas import tpu_sc as plsc

N = 10000
E = 160000
NGT = 29
D = 256
K = 128
NUM_LAYERS = 6

NC, NS, L = 2, 16, 16
NW = NC * NS
CH = 128
CPT = 40
EPT = CH * CPT
EP = EPT * NW
RPT = 632
NPAD = RPT * NS
PAD_DST = N + 5

def _mesh():
    return plsc.VectorSubcoreMesh(
        core_axis_name="c", subcore_axis_name="s",
        num_cores=NC, num_subcores=NS)


def _edge_body(g_hbm, c_hbm, src_hbm, dst_hbm, zero_hbm, out_hbm,
               sidx, didx, rows, crows, accum, sem):
    cid = lax.axis_index("c")
    sid = lax.axis_index("s")
    wid = cid * NS + sid
    pltpu.sync_copy(zero_hbm.at[pl.ds(sid * RPT, RPT)],
                    accum.at[pl.ds(sid * RPT, RPT)])
    plsc.subcore_barrier()

    def chunk_body(ch, carry):
        base = pl.multiple_of(wid * EPT + ch * CH, CH)
        pltpu.sync_copy(src_hbm.at[pl.ds(base, CH)], sidx)
        pltpu.sync_copy(dst_hbm.at[pl.ds(base, CH)], didx)
        pltpu.async_copy(g_hbm.at[sidx], rows, sem).wait()
        pltpu.sync_copy(c_hbm.at[pl.ds(base, CH)], crows)

        def row_body(r, c2):
            for j in range(K // L):
                x = rows[r, pl.ds(j * L, L)] + crows[r, pl.ds(j * L, L)]
                rows[r, pl.ds(j * L, L)] = jnp.maximum(x, x * 0.01)
            return c2
        lax.fori_loop(0, CH, row_body, 0)
        pltpu.sync_copy(rows, accum.at[didx], add=True)
        return carry
    lax.fori_loop(0, CPT, chunk_body, 0)
    plsc.subcore_barrier()
    pltpu.sync_copy(accum.at[pl.ds(sid * RPT, RPT)],
                    out_hbm.at[cid, pl.ds(sid * RPT, RPT)])


def _edge_call(*args):
    return pl.kernel(
        _edge_body,
        out_type=jax.ShapeDtypeStruct((NC, NPAD, K), jnp.float32),
        mesh=_mesh(),
        scratch_types=[
            pltpu.VMEM((CH,), jnp.int32),
            pltpu.VMEM((CH,), jnp.int32),
            pltpu.VMEM((CH, K), jnp.float32),
            pltpu.VMEM((CH, K), jnp.float32),
            pltpu.VMEM_SHARED((NPAD, K), jnp.float32),
            pltpu.SemaphoreType.DMA,
        ],
    )(*args)


def _deg_body(dst_hbm, zero_hbm, ones_hbm, out_hbm, didx, ones, dacc, sem):
    cid = lax.axis_index("c")
    sid = lax.axis_index("s")
    wid = cid * NS + sid
    pltpu.sync_copy(zero_hbm.at[pl.ds(sid * RPT, RPT)],
                    dacc.at[pl.ds(sid * RPT, RPT)])
    pltpu.sync_copy(ones_hbm, ones)
    plsc.subcore_barrier()

    def chunk_body(ch, carry):
        base = pl.multiple_of(wid * EPT + ch * CH, CH)
        pltpu.sync_copy(dst_hbm.at[pl.ds(base, CH)], didx)
        pltpu.sync_copy(ones, dacc.at[didx], add=True)
        return carry
    lax.fori_loop(0, CPT, chunk_body, 0)
    plsc.subcore_barrier()
    pltpu.sync_copy(dacc.at[pl.ds(sid * RPT, RPT)],
                    out_hbm.at[cid, pl.ds(sid * RPT, RPT)])


def _deg_call(*args):
    return pl.kernel(
        _deg_body,
        out_type=jax.ShapeDtypeStruct((NC, NPAD, K), jnp.float32),
        mesh=_mesh(),
        scratch_types=[
            pltpu.VMEM((CH,), jnp.int32),
            pltpu.VMEM((CH, K), jnp.float32),
            pltpu.VMEM_SHARED((NPAD, K), jnp.float32),
            pltpu.SemaphoreType.DMA,
        ],
    )(*args)


_PREC = lax.Precision.HIGHEST


def _embed0_body(gt_ref, emb_ref, w1a_ref, x_ref, g_ref):
    gt = gt_ref[0, 0, :]
    oh = (gt[:, None] == lax.broadcasted_iota(jnp.int32, (1, 32), 1)
          ).astype(jnp.float32)
    x = jnp.dot(oh, emb_ref[...], precision=_PREC)
    x_ref[...] = x
    g_ref[...] = jnp.dot(x, w1a_ref[...], precision=_PREC)


def _embed0(gt3, embp, w1a):
    rb = 1000
    return pl.pallas_call(
        _embed0_body,
        grid=(N // rb,),
        in_specs=[
            pl.BlockSpec((1, 1, rb), lambda i: (i, 0, 0)),
            pl.BlockSpec((32, D), lambda i: (0, 0)),
            pl.BlockSpec((D, K), lambda i: (0, 0)),
        ],
        out_specs=[
            pl.BlockSpec((rb, D), lambda i: (i, 0)),
            pl.BlockSpec((rb, K), lambda i: (i, 0)),
        ],
        out_shape=[
            jax.ShapeDtypeStruct((N, D), jnp.float32),
            jax.ShapeDtypeStruct((N, K), jnp.float32),
        ],
    )(gt3, embp, w1a)


def _cmat_body(ew_ref, w_ref, c_ref):
    ew = ew_ref[...]
    w = w_ref[...]
    c_ref[...] = (ew[:, 0:1] * w[0:1, :] + ew[:, 1:2] * w[1:2, :]
                  + ew[:, 2:3] * w[2:3, :])


def _cmat(ew_p, w1b):
    rb = 4096
    return pl.pallas_call(
        _cmat_body,
        grid=(EP // rb,),
        in_specs=[
            pl.BlockSpec((rb, 3), lambda i: (i, 0)),
            pl.BlockSpec((3, K), lambda i: (0, 0)),
        ],
        out_specs=pl.BlockSpec((rb, K), lambda i: (i, 0)),
        out_shape=jax.ShapeDtypeStruct((EP, K), jnp.float32),
    )(ew_p, w1b)


def _invdeg_body(d0_ref, d1_ref, o_ref):
    o_ref[...] = 1.0 / jnp.maximum(d0_ref[...] + d1_ref[...], 1.0)


def _invdeg(d0, d1):
    rb = 2000
    return pl.pallas_call(
        _invdeg_body,
        grid=(N // rb,),
        in_specs=[
            pl.BlockSpec((rb, L), lambda i: (i, 0)),
            pl.BlockSpec((rb, L), lambda i: (i, 0)),
        ],
        out_specs=pl.BlockSpec((rb, L), lambda i: (i, 0)),
        out_shape=jax.ShapeDtypeStruct((N, L), jnp.float32),
    )(d0, d1)


def _post_body(s0, s1, inv, x, w2a, w2b, b2, w1an, xn_ref, gn_ref):
    hN = (s0[...] + s1[...]) * inv[...][:, 0:1]
    h = (jnp.dot(x[...], w2a[...], precision=_PREC)
         + jnp.dot(hN, w2b[...], precision=_PREC) + b2[...])
    xn = jnp.maximum(h, 0.0)
    xn_ref[...] = xn
    gn_ref[...] = jnp.dot(xn, w1an[...], precision=_PREC)


def _post_last_body(s0, s1, inv, x, w2a, w2b, b2, h_ref):
    hN = (s0[...] + s1[...]) * inv[...][:, 0:1]
    h_ref[...] = (jnp.dot(x[...], w2a[...], precision=_PREC)
                  + jnp.dot(hN, w2b[...], precision=_PREC) + b2[...])


def _post(s0, s1, inv, x, w2a, w2b, b2r, w1an):
    rb = 400
    return pl.pallas_call(
        _post_body,
        grid=(N // rb,),
        in_specs=[
            pl.BlockSpec((rb, K), lambda i: (i, 0)),
            pl.BlockSpec((rb, K), lambda i: (i, 0)),
            pl.BlockSpec((rb, L), lambda i: (i, 0)),
            pl.BlockSpec((rb, D), lambda i: (i, 0)),
            pl.BlockSpec((D, D), lambda i: (0, 0)),
            pl.BlockSpec((K, D), lambda i: (0, 0)),
            pl.BlockSpec((1, D), lambda i: (0, 0)),
            pl.BlockSpec((D, K), lambda i: (0, 0)),
        ],
        out_specs=[
            pl.BlockSpec((rb, D), lambda i: (i, 0)),
            pl.BlockSpec((rb, K), lambda i: (i, 0)),
        ],
        out_shape=[
            jax.ShapeDtypeStruct((N, D), jnp.float32),
            jax.ShapeDtypeStruct((N, K), jnp.float32),
        ],
    )(s0, s1, inv, x, w2a, w2b, b2r, w1an)


def _post_last(s0, s1, inv, x, w2a, w2b, b2r):
    rb = 400
    return pl.pallas_call(
        _post_last_body,
        grid=(N // rb,),
        in_specs=[
            pl.BlockSpec((rb, K), lambda i: (i, 0)),
            pl.BlockSpec((rb, K), lambda i: (i, 0)),
            pl.BlockSpec((rb, L), lambda i: (i, 0)),
            pl.BlockSpec((rb, D), lambda i: (i, 0)),
            pl.BlockSpec((D, D), lambda i: (0, 0)),
            pl.BlockSpec((K, D), lambda i: (0, 0)),
            pl.BlockSpec((1, D), lambda i: (0, 0)),
        ],
        out_specs=pl.BlockSpec((rb, D), lambda i: (i, 0)),
        out_shape=jax.ShapeDtypeStruct((N, D), jnp.float32),
    )(s0, s1, inv, x, w2a, w2b, b2r)


def _head_body(x, wf1, bf1, wf2r, bf2, y_ref):
    t = jnp.maximum(jnp.dot(x[...], wf1[...], precision=_PREC) + bf1[...], 0.0)
    y_ref[...] = (jnp.sum(t * wf2r[...], axis=1, keepdims=True)
                  + bf2[0:1, 0:1])


def _head(h, wf1, bf1r, wf2r, bf2r):
    rb = 1000
    return pl.pallas_call(
        _head_body,
        grid=(N // rb,),
        in_specs=[
            pl.BlockSpec((rb, D), lambda i: (i, 0)),
            pl.BlockSpec((D, D), lambda i: (0, 0)),
            pl.BlockSpec((1, D), lambda i: (0, 0)),
            pl.BlockSpec((1, D), lambda i: (0, 0)),
            pl.BlockSpec((1, 1), lambda i: (0, 0)),
        ],
        out_specs=pl.BlockSpec((rb, 1), lambda i: (i, 0)),
        out_shape=jax.ShapeDtypeStruct((N, 1), jnp.float32),
    )(h, wf1, bf1r, wf2r, bf2r)


def kernel(gate_type, edge_index, edge_w, emb, W1, W2, b2, Wf1, bf1, Wf2, bf2):
    src = edge_index[0].astype(jnp.int32)
    dst = edge_index[1].astype(jnp.int32)
    src_p = jnp.concatenate([src, jnp.zeros((EP - E,), jnp.int32)])
    dst_p = jnp.concatenate([dst, jnp.full((EP - E,), PAD_DST, jnp.int32)])
    ew_p = jnp.pad(edge_w, ((0, EP - E), (0, 0)))

    W1a = W1[:, :D, :]
    W1b = W1[:, D:, :]
    W2a = W2[:, :D, :]
    W2b = W2[:, D:, :]
    gt3 = gate_type.astype(jnp.int32).reshape(N // 1000, 1, 1000)
    embp = jnp.pad(emb, ((0, 32 - NGT), (0, 0)))

    zeros_nk = jnp.zeros((NPAD, K), jnp.float32)
    ones_ck = jnp.ones((CH, K), jnp.float32)

    x, g = _embed0(gt3, embp, W1a[0])

    dparts = _deg_call(dst_p, zeros_nk, ones_ck)
    inv16 = _invdeg(dparts[0, :N, :L], dparts[1, :N, :L])

    h = None
    for i in range(NUM_LAYERS):
        c_i = _cmat(ew_p, W1b[i])
        parts = _edge_call(g, c_i, src_p, dst_p, zeros_nk)
        s0 = parts[0, :N]
        s1 = parts[1, :N]
        b2r = b2[i].reshape(1, D)
        if i < NUM_LAYERS - 1:
            x, g = _post(s0, s1, inv16, x, W2a[i], W2b[i], b2r, W1a[i + 1])
        else:
            h = _post_last(s0, s1, inv16, x, W2a[i], W2b[i], b2r)

    return _head(h, Wf1, bf1.reshape(1, D), Wf2.reshape(1, D),
                 bf2.reshape(1, 1))


# speedup vs baseline: 2.9632x; 1.7279x over previous
import functools

import jax
import jax.numpy as jnp
from jax import lax
from jax.experimental import pallas as pl
from jax.experimental.pallas import tpu as pltpu
from jax.experimental.pallas import tpu_sc as plsc

N = 10000
E = 160000
NGT = 29
D = 256
K = 128
NUM_LAYERS = 6

NC, NS, L = 2, 16, 16
NW = NC * NS
CH = 128
CPT = 40
EPT = CH * CPT
EP = EPT * NW
RPT = 632
NPAD = RPT * NS
PAD_DST = N + 5


def _mesh():
    return plsc.VectorSubcoreMesh(
        core_axis_name="c", subcore_axis_name="s",
        num_cores=NC, num_subcores=NS)


def _edge_body(g_hbm, ew_hbm, w1b_hbm, src_hbm, dst_hbm, zero_hbm, out_hbm,
               sidx, didx, grows0, grows1, ewb0, ewb1, wbuf,
               accum, sem_i, sem_g0, sem_g1, sem_e0, sem_e1, sem_s0, sem_s1):
    cid = lax.axis_index("c")
    sid = lax.axis_index("s")
    wid = cid * NS + sid
    grows = (grows0, grows1)
    ewb = (ewb0, ewb1)
    sem_g = (sem_g0, sem_g1)
    sem_e = (sem_e0, sem_e1)
    sem_s = (sem_s0, sem_s1)
    ebase = wid * EPT

    pltpu.async_copy(src_hbm.at[wid], sidx, sem_i)
    pltpu.async_copy(dst_hbm.at[wid], didx, sem_i)
    pltpu.sync_copy(w1b_hbm, wbuf)
    pltpu.sync_copy(zero_hbm.at[pl.ds(sid * RPT, RPT)],
                    accum.at[pl.ds(sid * RPT, RPT)])
    pltpu.make_async_copy(src_hbm.at[wid], sidx, sem_i).wait()
    pltpu.make_async_copy(dst_hbm.at[wid], didx, sem_i).wait()
    plsc.subcore_barrier()

    def fire_gather(ch, b):
        pltpu.async_copy(g_hbm.at[sidx.at[ch]], grows[b], sem_g[b])

    def wait_gather(ch, b):
        pltpu.make_async_copy(g_hbm.at[sidx.at[ch]], grows[b], sem_g[b]).wait()

    def fire_ew(ch, b):
        pltpu.async_copy(ew_hbm.at[:, pl.ds(ebase + ch * CH, CH)], ewb[b],
                         sem_e[b])

    def wait_ew(ch, b):
        pltpu.make_async_copy(ew_hbm.at[:, pl.ds(ebase + ch * CH, CH)],
                              ewb[b], sem_e[b]).wait()

    def fire_scatter(ch, b):
        pltpu.async_copy(grows[b], accum.at[didx.at[ch]], sem_s[b], add=True)

    def wait_scatter(ch, b):
        pltpu.make_async_copy(grows[b], accum.at[didx.at[ch]],
                              sem_s[b]).wait()

    def compute(b):
        g_r, e_r = grows[b], ewb[b]
        w = [[wbuf[k, pl.ds(j * L, L)] for j in range(K // L)]
             for k in range(3)]

        def group_body(gi, carry):
            base16 = gi * L
            e0v = e_r[0, pl.ds(base16, L)]
            e1v = e_r[1, pl.ds(base16, L)]
            e2v = e_r[2, pl.ds(base16, L)]
            for p in range(L):
                row = base16 + p
                s0 = jnp.full((L,), e0v[p], jnp.float32)
                s1 = jnp.full((L,), e1v[p], jnp.float32)
                s2 = jnp.full((L,), e2v[p], jnp.float32)
                for j in range(K // L):
                    sl = pl.ds(j * L, L)
                    x = g_r[row, sl] + (s0 * w[0][j] + s1 * w[1][j]
                                        + s2 * w[2][j])
                    g_r[row, sl] = jnp.maximum(x, x * 0.01)
            return carry
        lax.fori_loop(0, CH // L, group_body, 0)

    fire_ew(0, 0)
    fire_ew(1, 1)
    fire_gather(0, 0)

    def outer(it, carry):
        for b in (0, 1):
            ch = it * 2 + b

            @pl.when(ch >= 1)
            def _():
                wait_scatter(ch - 1, 1 - b)

            @pl.when(ch + 1 < CPT)
            def _():
                fire_gather(ch + 1, 1 - b)
            wait_gather(ch, b)
            wait_ew(ch, b)
            compute(b)
            fire_scatter(ch, b)

            @pl.when(ch + 2 < CPT)
            def _():
                fire_ew(ch + 2, b)
        return carry
    lax.fori_loop(0, CPT // 2, outer, 0)
    wait_scatter(CPT - 1, 1)
    plsc.subcore_barrier()
    pltpu.sync_copy(accum.at[pl.ds(sid * RPT, RPT)],
                    out_hbm.at[cid, pl.ds(sid * RPT, RPT)])


def _edge_call(*args):
    return pl.kernel(
        _edge_body,
        out_type=jax.ShapeDtypeStruct((NC, NPAD, K), jnp.float32),
        mesh=_mesh(),
        scratch_types=[
            pltpu.VMEM((CPT, CH), jnp.int32),
            pltpu.VMEM((CPT, CH), jnp.int32),
            pltpu.VMEM((CH, K), jnp.float32),
            pltpu.VMEM((CH, K), jnp.float32),
            pltpu.VMEM((3, CH), jnp.float32),
            pltpu.VMEM((3, CH), jnp.float32),
            pltpu.VMEM((3, K), jnp.float32),
            pltpu.VMEM_SHARED((NPAD, K), jnp.float32),
            pltpu.SemaphoreType.DMA,
            pltpu.SemaphoreType.DMA,
            pltpu.SemaphoreType.DMA,
            pltpu.SemaphoreType.DMA,
            pltpu.SemaphoreType.DMA,
            pltpu.SemaphoreType.DMA,
            pltpu.SemaphoreType.DMA,
        ],
    )(*args)


def _deg_body(dst_hbm, zero_hbm, ones_hbm, out_hbm, didx, ones, dacc,
              sem_i, sem_s):
    cid = lax.axis_index("c")
    sid = lax.axis_index("s")
    wid = cid * NS + sid
    pltpu.async_copy(dst_hbm.at[wid], didx, sem_i)
    pltpu.sync_copy(zero_hbm.at[pl.ds(sid * RPT, RPT)],
                    dacc.at[pl.ds(sid * RPT, RPT)])
    pltpu.sync_copy(ones_hbm, ones)
    pltpu.make_async_copy(dst_hbm.at[wid], didx, sem_i).wait()
    plsc.subcore_barrier()

    def fire_body(ch, carry):
        pltpu.async_copy(ones, dacc.at[didx.at[ch]], sem_s, add=True)
        return carry
    lax.fori_loop(0, CPT, fire_body, 0)

    def drain_body(ch, carry):
        pltpu.make_async_copy(ones, dacc.at[didx.at[ch]], sem_s).wait()
        return carry
    lax.fori_loop(0, CPT, drain_body, 0)
    plsc.subcore_barrier()
    pltpu.sync_copy(dacc.at[pl.ds(sid * RPT, RPT)],
                    out_hbm.at[cid, pl.ds(sid * RPT, RPT)])


def _deg_call(*args):
    return pl.kernel(
        _deg_body,
        out_type=jax.ShapeDtypeStruct((NC, NPAD, K), jnp.float32),
        mesh=_mesh(),
        scratch_types=[
            pltpu.VMEM((CPT, CH), jnp.int32),
            pltpu.VMEM((CH, K), jnp.float32),
            pltpu.VMEM_SHARED((NPAD, K), jnp.float32),
            pltpu.SemaphoreType.DMA,
            pltpu.SemaphoreType.DMA,
        ],
    )(*args)


_PREC = lax.Precision.HIGHEST


def _embed0_body(gt_ref, emb_ref, w1a_ref, x_ref, g_ref):
    gt = gt_ref[0, 0, :]
    oh = (gt[:, None] == lax.broadcasted_iota(jnp.int32, (1, 32), 1)
          ).astype(jnp.float32)
    x = jnp.dot(oh, emb_ref[...], precision=_PREC)
    x_ref[...] = x
    g_ref[...] = jnp.dot(x, w1a_ref[...], precision=_PREC)


def _embed0(gt3, embp, w1a):
    rb = 1000
    return pl.pallas_call(
        _embed0_body,
        grid=(N // rb,),
        in_specs=[
            pl.BlockSpec((1, 1, rb), lambda i: (i, 0, 0)),
            pl.BlockSpec((32, D), lambda i: (0, 0)),
            pl.BlockSpec((D, K), lambda i: (0, 0)),
        ],
        out_specs=[
            pl.BlockSpec((rb, D), lambda i: (i, 0)),
            pl.BlockSpec((rb, K), lambda i: (i, 0)),
        ],
        out_shape=[
            jax.ShapeDtypeStruct((N, D), jnp.float32),
            jax.ShapeDtypeStruct((N, K), jnp.float32),
        ],
    )(gt3, embp, w1a)


def _invdeg_body(d0_ref, d1_ref, o_ref):
    o_ref[...] = 1.0 / jnp.maximum(d0_ref[...] + d1_ref[...], 1.0)


def _invdeg(d0, d1):
    rb = 2000
    return pl.pallas_call(
        _invdeg_body,
        grid=(N // rb,),
        in_specs=[
            pl.BlockSpec((rb, L), lambda i: (i, 0)),
            pl.BlockSpec((rb, L), lambda i: (i, 0)),
        ],
        out_specs=pl.BlockSpec((rb, L), lambda i: (i, 0)),
        out_shape=jax.ShapeDtypeStruct((N, L), jnp.float32),
    )(d0, d1)


def _post_body(s0, s1, inv, x, w2a, w2b, b2, w1an, xn_ref, gn_ref):
    hN = (s0[...] + s1[...]) * inv[...][:, 0:1]
    h = (jnp.dot(x[...], w2a[...], precision=_PREC)
         + jnp.dot(hN, w2b[...], precision=_PREC) + b2[...])
    xn = jnp.maximum(h, 0.0)
    xn_ref[...] = xn
    gn_ref[...] = jnp.dot(xn, w1an[...], precision=_PREC)


def _post_last_body(s0, s1, inv, x, w2a, w2b, b2, h_ref):
    hN = (s0[...] + s1[...]) * inv[...][:, 0:1]
    h_ref[...] = (jnp.dot(x[...], w2a[...], precision=_PREC)
                  + jnp.dot(hN, w2b[...], precision=_PREC) + b2[...])


def _post(s0, s1, inv, x, w2a, w2b, b2r, w1an):
    rb = 400
    return pl.pallas_call(
        _post_body,
        grid=(N // rb,),
        in_specs=[
            pl.BlockSpec((rb, K), lambda i: (i, 0)),
            pl.BlockSpec((rb, K), lambda i: (i, 0)),
            pl.BlockSpec((rb, L), lambda i: (i, 0)),
            pl.BlockSpec((rb, D), lambda i: (i, 0)),
            pl.BlockSpec((D, D), lambda i: (0, 0)),
            pl.BlockSpec((K, D), lambda i: (0, 0)),
            pl.BlockSpec((1, D), lambda i: (0, 0)),
            pl.BlockSpec((D, K), lambda i: (0, 0)),
        ],
        out_specs=[
            pl.BlockSpec((rb, D), lambda i: (i, 0)),
            pl.BlockSpec((rb, K), lambda i: (i, 0)),
        ],
        out_shape=[
            jax.ShapeDtypeStruct((N, D), jnp.float32),
            jax.ShapeDtypeStruct((N, K), jnp.float32),
        ],
    )(s0, s1, inv, x, w2a, w2b, b2r, w1an)


def _post_last(s0, s1, inv, x, w2a, w2b, b2r):
    rb = 400
    return pl.pallas_call(
        _post_last_body,
        grid=(N // rb,),
        in_specs=[
            pl.BlockSpec((rb, K), lambda i: (i, 0)),
            pl.BlockSpec((rb, K), lambda i: (i, 0)),
            pl.BlockSpec((rb, L), lambda i: (i, 0)),
            pl.BlockSpec((rb, D), lambda i: (i, 0)),
            pl.BlockSpec((D, D), lambda i: (0, 0)),
            pl.BlockSpec((K, D), lambda i: (0, 0)),
            pl.BlockSpec((1, D), lambda i: (0, 0)),
        ],
        out_specs=pl.BlockSpec((rb, D), lambda i: (i, 0)),
        out_shape=jax.ShapeDtypeStruct((N, D), jnp.float32),
    )(s0, s1, inv, x, w2a, w2b, b2r)


def _head_body(x, wf1, bf1, wf2r, bf2, y_ref):
    t = jnp.maximum(jnp.dot(x[...], wf1[...], precision=_PREC) + bf1[...], 0.0)
    y_ref[...] = (jnp.sum(t * wf2r[...], axis=1, keepdims=True)
                  + bf2[0:1, 0:1])


def _head(h, wf1, bf1r, wf2r, bf2r):
    rb = 1000
    return pl.pallas_call(
        _head_body,
        grid=(N // rb,),
        in_specs=[
            pl.BlockSpec((rb, D), lambda i: (i, 0)),
            pl.BlockSpec((D, D), lambda i: (0, 0)),
            pl.BlockSpec((1, D), lambda i: (0, 0)),
            pl.BlockSpec((1, D), lambda i: (0, 0)),
            pl.BlockSpec((1, 1), lambda i: (0, 0)),
        ],
        out_specs=pl.BlockSpec((rb, 1), lambda i: (i, 0)),
        out_shape=jax.ShapeDtypeStruct((N, 1), jnp.float32),
    )(h, wf1, bf1r, wf2r, bf2r)


def kernel(gate_type, edge_index, edge_w, emb, W1, W2, b2, Wf1, bf1, Wf2, bf2):
    src = edge_index[0].astype(jnp.int32)
    dst = edge_index[1].astype(jnp.int32)
    src_p = jnp.concatenate(
        [src, jnp.zeros((EP - E,), jnp.int32)]).reshape(NW, CPT, CH)
    dst_p = jnp.concatenate(
        [dst, jnp.full((EP - E,), PAD_DST, jnp.int32)]).reshape(NW, CPT, CH)
    ew_t = jnp.pad(edge_w, ((0, EP - E), (0, 0))).T

    W1a = W1[:, :D, :]
    W1b = W1[:, D:, :]
    W2a = W2[:, :D, :]
    W2b = W2[:, D:, :]
    gt3 = gate_type.astype(jnp.int32).reshape(N // 1000, 1, 1000)
    embp = jnp.pad(emb, ((0, 32 - NGT), (0, 0)))

    zeros_nk = jnp.zeros((NPAD, K), jnp.float32)
    ones_ck = jnp.ones((CH, K), jnp.float32)

    x, g = _embed0(gt3, embp, W1a[0])

    dparts = _deg_call(dst_p, zeros_nk, ones_ck)
    inv16 = _invdeg(dparts[0, :N, :L], dparts[1, :N, :L])

    h = None
    for i in range(NUM_LAYERS):
        parts = _edge_call(g, ew_t, W1b[i], src_p, dst_p, zeros_nk)
        s0 = parts[0, :N]
        s1 = parts[1, :N]
        b2r = b2[i].reshape(1, D)
        if i < NUM_LAYERS - 1:
            x, g = _post(s0, s1, inv16, x, W2a[i], W2b[i], b2r, W1a[i + 1])
        else:
            h = _post_last(s0, s1, inv16, x, W2a[i], W2b[i], b2r)

    return _head(h, Wf1, bf1.reshape(1, D), Wf2.reshape(1, D),
                 bf2.reshape(1, 1))


# speedup vs baseline: 3.1694x; 1.0696x over previous
import functools

import jax
import jax.numpy as jnp
from jax import lax
from jax.experimental import pallas as pl
from jax.experimental.pallas import tpu as pltpu
from jax.experimental.pallas import tpu_sc as plsc

N = 10000
E = 160000
NGT = 29
D = 256
K = 128
NUM_LAYERS = 6

NC, NS, L = 2, 16, 16
NW = NC * NS
CH = 128
CPT = 40
EPT = CH * CPT
EP = EPT * NW
RPT = 632
NPAD = RPT * NS
PAD_DST = N + 5


def _mesh():
    return plsc.VectorSubcoreMesh(
        core_axis_name="c", subcore_axis_name="s",
        num_cores=NC, num_subcores=NS)


def _edge_body(g_hbm, ew_hbm, w1b_hbm, src_hbm, dst_hbm, zero_hbm, out_hbm,
               sidx, didx, grows0, grows1, ewb0, ewb1, wbuf,
               accum, sem_i, sem_g0, sem_g1, sem_e0, sem_e1, sem_s0, sem_s1):
    cid = lax.axis_index("c")
    sid = lax.axis_index("s")
    wid = cid * NS + sid
    grows = (grows0, grows1)
    ewb = (ewb0, ewb1)
    sem_g = (sem_g0, sem_g1)
    sem_e = (sem_e0, sem_e1)
    sem_s = (sem_s0, sem_s1)
    ebase = wid * EPT

    pltpu.async_copy(src_hbm.at[wid], sidx, sem_i)
    pltpu.async_copy(dst_hbm.at[wid], didx, sem_i)
    pltpu.sync_copy(w1b_hbm, wbuf)
    pltpu.sync_copy(zero_hbm.at[pl.ds(sid * RPT, RPT)],
                    accum.at[pl.ds(sid * RPT, RPT)])
    pltpu.make_async_copy(src_hbm.at[wid], sidx, sem_i).wait()
    pltpu.make_async_copy(dst_hbm.at[wid], didx, sem_i).wait()
    plsc.subcore_barrier()

    def fire_gather(ch, b):
        pltpu.async_copy(g_hbm.at[sidx.at[ch]], grows[b], sem_g[b])

    def wait_gather(ch, b):
        pltpu.make_async_copy(g_hbm.at[sidx.at[ch]], grows[b], sem_g[b]).wait()

    def fire_ew(ch, b):
        pltpu.async_copy(ew_hbm.at[:, pl.ds(ebase + ch * CH, CH)], ewb[b],
                         sem_e[b])

    def wait_ew(ch, b):
        pltpu.make_async_copy(ew_hbm.at[:, pl.ds(ebase + ch * CH, CH)],
                              ewb[b], sem_e[b]).wait()

    def fire_scatter(ch, b):
        pltpu.async_copy(grows[b], accum.at[didx.at[ch]], sem_s[b], add=True)

    def wait_scatter(ch, b):
        pltpu.make_async_copy(grows[b], accum.at[didx.at[ch]],
                              sem_s[b]).wait()

    def compute(b):
        g_r, e_r = grows[b], ewb[b]
        w = [[wbuf[k, pl.ds(j * L, L)] for j in range(K // L)]
             for k in range(3)]

        def group_body(gi, carry):
            base16 = gi * L
            e0v = e_r[0, pl.ds(base16, L)]
            e1v = e_r[1, pl.ds(base16, L)]
            e2v = e_r[2, pl.ds(base16, L)]
            for p in range(L):
                row = base16 + p
                s0 = jnp.full((L,), e0v[p], jnp.float32)
                s1 = jnp.full((L,), e1v[p], jnp.float32)
                s2 = jnp.full((L,), e2v[p], jnp.float32)
                for j in range(K // L):
                    sl = pl.ds(j * L, L)
                    x = g_r[row, sl] + (s0 * w[0][j] + s1 * w[1][j]
                                        + s2 * w[2][j])
                    g_r[row, sl] = jnp.maximum(x, x * 0.01)
            return carry
        lax.fori_loop(0, CH // L, group_body, 0)

    fire_ew(0, 0)
    fire_ew(1, 1)
    fire_gather(0, 0)

    def outer(it, carry):
        for b in (0, 1):
            ch = it * 2 + b

            @pl.when(ch >= 1)
            def _():
                wait_scatter(ch - 1, 1 - b)

            @pl.when(ch + 1 < CPT)
            def _():
                fire_gather(ch + 1, 1 - b)
            wait_gather(ch, b)
            wait_ew(ch, b)
            compute(b)
            fire_scatter(ch, b)

            @pl.when(ch + 2 < CPT)
            def _():
                fire_ew(ch + 2, b)
        return carry
    lax.fori_loop(0, CPT // 2, outer, 0)
    wait_scatter(CPT - 1, 1)
    plsc.subcore_barrier()
    pltpu.sync_copy(accum.at[pl.ds(sid * RPT, RPT)],
                    out_hbm.at[cid, pl.ds(sid * RPT, RPT)])


def _edge_call(*args):
    return pl.kernel(
        _edge_body,
        out_type=jax.ShapeDtypeStruct((NC, NPAD, K), jnp.float32),
        mesh=_mesh(),
        scratch_types=[
            pltpu.VMEM((CPT, CH), jnp.int32),
            pltpu.VMEM((CPT, CH), jnp.int32),
            pltpu.VMEM((CH, K), jnp.float32),
            pltpu.VMEM((CH, K), jnp.float32),
            pltpu.VMEM((3, CH), jnp.float32),
            pltpu.VMEM((3, CH), jnp.float32),
            pltpu.VMEM((3, K), jnp.float32),
            pltpu.VMEM_SHARED((NPAD, K), jnp.float32),
            pltpu.SemaphoreType.DMA,
            pltpu.SemaphoreType.DMA,
            pltpu.SemaphoreType.DMA,
            pltpu.SemaphoreType.DMA,
            pltpu.SemaphoreType.DMA,
            pltpu.SemaphoreType.DMA,
            pltpu.SemaphoreType.DMA,
        ],
    )(*args)


def _deg_body(dst_hbm, zero_hbm, ones_hbm, out_hbm, didx, ones, dacc,
              sem_i, sem_s):
    cid = lax.axis_index("c")
    sid = lax.axis_index("s")
    wid = cid * NS + sid
    pltpu.async_copy(dst_hbm.at[wid], didx, sem_i)
    pltpu.sync_copy(zero_hbm.at[pl.ds(sid * RPT, RPT)],
                    dacc.at[pl.ds(sid * RPT, RPT)])
    pltpu.sync_copy(ones_hbm, ones)
    pltpu.make_async_copy(dst_hbm.at[wid], didx, sem_i).wait()
    plsc.subcore_barrier()

    def fire_body(ch, carry):
        pltpu.async_copy(ones, dacc.at[didx.at[ch]], sem_s, add=True)
        return carry
    lax.fori_loop(0, CPT, fire_body, 0)

    def drain_body(ch, carry):
        pltpu.make_async_copy(ones, dacc.at[didx.at[ch]], sem_s).wait()
        return carry
    lax.fori_loop(0, CPT, drain_body, 0)
    plsc.subcore_barrier()
    pltpu.sync_copy(dacc.at[pl.ds(sid * RPT, RPT)],
                    out_hbm.at[cid, pl.ds(sid * RPT, RPT)])


def _deg_call(*args):
    return pl.kernel(
        _deg_body,
        out_type=jax.ShapeDtypeStruct((NC, NPAD, K), jnp.float32),
        mesh=_mesh(),
        scratch_types=[
            pltpu.VMEM((CPT, CH), jnp.int32),
            pltpu.VMEM((CH, K), jnp.float32),
            pltpu.VMEM_SHARED((NPAD, K), jnp.float32),
            pltpu.SemaphoreType.DMA,
            pltpu.SemaphoreType.DMA,
        ],
    )(*args)


_PREC = lax.Precision.DEFAULT


def _embed0_body(gt_ref, emb_ref, w1a_ref, x_ref, g_ref):
    gt = gt_ref[0, 0, :]
    oh = (gt[:, None] == lax.broadcasted_iota(jnp.int32, (1, 32), 1)
          ).astype(jnp.float32)
    x = jnp.dot(oh, emb_ref[...], precision=_PREC)
    x_ref[...] = x
    g_ref[...] = jnp.dot(x, w1a_ref[...], precision=_PREC)


def _embed0(gt3, embp, w1a):
    rb = 1000
    return pl.pallas_call(
        _embed0_body,
        grid=(N // rb,),
        in_specs=[
            pl.BlockSpec((1, 1, rb), lambda i: (i, 0, 0)),
            pl.BlockSpec((32, D), lambda i: (0, 0)),
            pl.BlockSpec((D, K), lambda i: (0, 0)),
        ],
        out_specs=[
            pl.BlockSpec((rb, D), lambda i: (i, 0)),
            pl.BlockSpec((rb, K), lambda i: (i, 0)),
        ],
        out_shape=[
            jax.ShapeDtypeStruct((N, D), jnp.float32),
            jax.ShapeDtypeStruct((N, K), jnp.float32),
        ],
    )(gt3, embp, w1a)


def _invdeg_body(d0_ref, d1_ref, o_ref):
    o_ref[...] = 1.0 / jnp.maximum(d0_ref[...] + d1_ref[...], 1.0)


def _invdeg(d0, d1):
    rb = 2000
    return pl.pallas_call(
        _invdeg_body,
        grid=(N // rb,),
        in_specs=[
            pl.BlockSpec((rb, L), lambda i: (i, 0)),
            pl.BlockSpec((rb, L), lambda i: (i, 0)),
        ],
        out_specs=pl.BlockSpec((rb, L), lambda i: (i, 0)),
        out_shape=jax.ShapeDtypeStruct((N, L), jnp.float32),
    )(d0, d1)


def _post_body(s0, s1, inv, x, w2a, w2b, b2, w1an, xn_ref, gn_ref):
    hN = (s0[...] + s1[...]) * inv[...][:, 0:1]
    h = (jnp.dot(x[...], w2a[...], precision=_PREC)
         + jnp.dot(hN, w2b[...], precision=_PREC) + b2[...])
    xn = jnp.maximum(h, 0.0)
    xn_ref[...] = xn
    gn_ref[...] = jnp.dot(xn, w1an[...], precision=_PREC)


def _post_last_body(s0, s1, inv, x, w2a, w2b, b2, h_ref):
    hN = (s0[...] + s1[...]) * inv[...][:, 0:1]
    h_ref[...] = (jnp.dot(x[...], w2a[...], precision=_PREC)
                  + jnp.dot(hN, w2b[...], precision=_PREC) + b2[...])


def _post(s0, s1, inv, x, w2a, w2b, b2r, w1an):
    rb = 400
    return pl.pallas_call(
        _post_body,
        grid=(N // rb,),
        in_specs=[
            pl.BlockSpec((rb, K), lambda i: (i, 0)),
            pl.BlockSpec((rb, K), lambda i: (i, 0)),
            pl.BlockSpec((rb, L), lambda i: (i, 0)),
            pl.BlockSpec((rb, D), lambda i: (i, 0)),
            pl.BlockSpec((D, D), lambda i: (0, 0)),
            pl.BlockSpec((K, D), lambda i: (0, 0)),
            pl.BlockSpec((1, D), lambda i: (0, 0)),
            pl.BlockSpec((D, K), lambda i: (0, 0)),
        ],
        out_specs=[
            pl.BlockSpec((rb, D), lambda i: (i, 0)),
            pl.BlockSpec((rb, K), lambda i: (i, 0)),
        ],
        out_shape=[
            jax.ShapeDtypeStruct((N, D), jnp.float32),
            jax.ShapeDtypeStruct((N, K), jnp.float32),
        ],
    )(s0, s1, inv, x, w2a, w2b, b2r, w1an)


def _post_last(s0, s1, inv, x, w2a, w2b, b2r):
    rb = 400
    return pl.pallas_call(
        _post_last_body,
        grid=(N // rb,),
        in_specs=[
            pl.BlockSpec((rb, K), lambda i: (i, 0)),
            pl.BlockSpec((rb, K), lambda i: (i, 0)),
            pl.BlockSpec((rb, L), lambda i: (i, 0)),
            pl.BlockSpec((rb, D), lambda i: (i, 0)),
            pl.BlockSpec((D, D), lambda i: (0, 0)),
            pl.BlockSpec((K, D), lambda i: (0, 0)),
            pl.BlockSpec((1, D), lambda i: (0, 0)),
        ],
        out_specs=pl.BlockSpec((rb, D), lambda i: (i, 0)),
        out_shape=jax.ShapeDtypeStruct((N, D), jnp.float32),
    )(s0, s1, inv, x, w2a, w2b, b2r)


def _head_body(x, wf1, bf1, wf2r, bf2, y_ref):
    t = jnp.maximum(jnp.dot(x[...], wf1[...], precision=_PREC) + bf1[...], 0.0)
    y_ref[...] = (jnp.sum(t * wf2r[...], axis=1, keepdims=True)
                  + bf2[0:1, 0:1])


def _head(h, wf1, bf1r, wf2r, bf2r):
    rb = 1000
    return pl.pallas_call(
        _head_body,
        grid=(N // rb,),
        in_specs=[
            pl.BlockSpec((rb, D), lambda i: (i, 0)),
            pl.BlockSpec((D, D), lambda i: (0, 0)),
            pl.BlockSpec((1, D), lambda i: (0, 0)),
            pl.BlockSpec((1, D), lambda i: (0, 0)),
            pl.BlockSpec((1, 1), lambda i: (0, 0)),
        ],
        out_specs=pl.BlockSpec((rb, 1), lambda i: (i, 0)),
        out_shape=jax.ShapeDtypeStruct((N, 1), jnp.float32),
    )(h, wf1, bf1r, wf2r, bf2r)


def kernel(gate_type, edge_index, edge_w, emb, W1, W2, b2, Wf1, bf1, Wf2, bf2):
    src = edge_index[0].astype(jnp.int32)
    dst = edge_index[1].astype(jnp.int32)
    src_p = jnp.concatenate(
        [src, jnp.zeros((EP - E,), jnp.int32)]).reshape(NW, CPT, CH)
    pad_dst = N + jnp.arange(EP - E, dtype=jnp.int32) % (NPAD - N)
    dst_p = jnp.concatenate([dst, pad_dst]).reshape(NW, CPT, CH)
    ew_t = jnp.pad(edge_w, ((0, EP - E), (0, 0))).T

    W1a = W1[:, :D, :]
    W1b = W1[:, D:, :]
    W2a = W2[:, :D, :]
    W2b = W2[:, D:, :]
    gt3 = gate_type.astype(jnp.int32).reshape(N // 1000, 1, 1000)
    embp = jnp.pad(emb, ((0, 32 - NGT), (0, 0)))

    zeros_nk = jnp.zeros((NPAD, K), jnp.float32)
    ones_ck = jnp.ones((CH, K), jnp.float32)

    x, g = _embed0(gt3, embp, W1a[0])

    dparts = _deg_call(dst_p, zeros_nk, ones_ck)
    inv16 = _invdeg(dparts[0, :N, :L], dparts[1, :N, :L])

    h = None
    for i in range(NUM_LAYERS):
        parts = _edge_call(g, ew_t, W1b[i], src_p, dst_p, zeros_nk)
        s0 = parts[0, :N]
        s1 = parts[1, :N]
        b2r = b2[i].reshape(1, D)
        if i < NUM_LAYERS - 1:
            x, g = _post(s0, s1, inv16, x, W2a[i], W2b[i], b2r, W1a[i + 1])
        else:
            h = _post_last(s0, s1, inv16, x, W2a[i], W2b[i], b2r)

    return _head(h, Wf1, bf1.reshape(1, D), Wf2.reshape(1, D),
                 bf2.reshape(1, 1))
